# baseline (speedup 1.0000x reference)
import functools

import jax
import jax.numpy as jnp
from jax import lax
from jax.experimental import pallas as pl
from jax.experimental.pallas import tpu as pltpu
from jax.experimental.pallas import tpu_sc as plsc

N = 10000
D = 128
H = 128
C = 40

NC = 2
NS = 16
NW = NC * NS
NP = 10240
RPT = NP // NS

W1P = 128
W2P = 48
CW = 8

EB = 128
ECH = 8
NB0 = 136
NB1 = 24

BM = 2000


def _tc_layer1(x, W1_l, W1_r, b1):
    def body(x_ref, wl_ref, wr_ref, b_ref, y1_ref, xr_ref):
        xb = x_ref[...]
        y1_ref[...] = jnp.dot(xb, wl_ref[...], preferred_element_type=jnp.float32)
        xr_ref[...] = jnp.dot(xb, wr_ref[...], preferred_element_type=jnp.float32) + b_ref[...]

    return pl.pallas_call(
        body,
        grid=(N // BM,),
        in_specs=[
            pl.BlockSpec((BM, D), lambda i: (i, 0)),
            pl.BlockSpec((D, H), lambda i: (0, 0)),
            pl.BlockSpec((D, H), lambda i: (0, 0)),
            pl.BlockSpec((1, H), lambda i: (0, 0)),
        ],
        out_specs=[
            pl.BlockSpec((BM, W1P), lambda i: (i, 0)),
            pl.BlockSpec((BM, H), lambda i: (i, 0)),
        ],
        out_shape=[
            jax.ShapeDtypeStruct((N, W1P), jnp.float32),
            jax.ShapeDtypeStruct((N, H), jnp.float32),
        ],
    )(x, W1_l, W1_r, b1.reshape(1, H))


def _tc_layer2(part1, cntp, xr1, W2_l, W2_r, b2):
    def body(p_ref, c_ref, xr_ref, wl_ref, wr_ref, b_ref, y2_ref, hr_ref, inv_ref):
        p = p_ref[0] + p_ref[1]
        cnt = (c_ref[0] + c_ref[1])[:, :1]
        inv = 1.0 / jnp.maximum(cnt, 1.0)
        h = jnp.maximum(p * inv + xr_ref[...], 0.0)
        y2_ref[:, :C] = jnp.dot(h, wl_ref[...], preferred_element_type=jnp.float32)
        y2_ref[:, C:] = jnp.zeros((BM, W2P - C), jnp.float32)
        hr_ref[...] = jnp.dot(h, wr_ref[...], preferred_element_type=jnp.float32) + b_ref[...]
        inv_ref[...] = jnp.broadcast_to(inv, (BM, 8))

    return pl.pallas_call(
        body,
        grid=(N // BM,),
        in_specs=[
            pl.BlockSpec((NC, BM, W1P), lambda i: (0, i, 0)),
            pl.BlockSpec((NC, BM, CW), lambda i: (0, i, 0)),
            pl.BlockSpec((BM, H), lambda i: (i, 0)),
            pl.BlockSpec((H, C), lambda i: (0, 0)),
            pl.BlockSpec((H, C), lambda i: (0, 0)),
            pl.BlockSpec((1, C), lambda i: (0, 0)),
        ],
        out_specs=[
            pl.BlockSpec((BM, W2P), lambda i: (i, 0)),
            pl.BlockSpec((BM, C), lambda i: (i, 0)),
            pl.BlockSpec((BM, 8), lambda i: (i, 0)),
        ],
        out_shape=[
            jax.ShapeDtypeStruct((N, W2P), jnp.float32),
            jax.ShapeDtypeStruct((N, C), jnp.float32),
            jax.ShapeDtypeStruct((N, 8), jnp.float32),
        ],
    )(part1, cntp, xr1, W2_l, W2_r, b2.reshape(1, C))


def _tc_out(part2, inv, hr2):
    def body(p_ref, inv_ref, hr_ref, o_ref):
        p = p_ref[0] + p_ref[1]
        o_ref[...] = p[:, :C] * inv_ref[:, :1] + hr_ref[...]

    return pl.pallas_call(
        body,
        grid=(N // BM,),
        in_specs=[
            pl.BlockSpec((NC, BM, W2P), lambda i: (0, i, 0)),
            pl.BlockSpec((BM, 8), lambda i: (i, 0)),
            pl.BlockSpec((BM, C), lambda i: (i, 0)),
        ],
        out_specs=pl.BlockSpec((BM, C), lambda i: (i, 0)),
        out_shape=jax.ShapeDtypeStruct((N, C), jnp.float32),
    )(part2, inv, hr2)


def _seg_sum(table, src2, dst2, zeros, zcnt, ones, width, with_counts, tiled=False):
    mesh = plsc.VectorSubcoreMesh(
        core_axis_name="c", subcore_axis_name="s", num_cores=NC, num_subcores=NS)

    out_type = [jax.ShapeDtypeStruct((NC, N, width), jnp.float32)]
    scratch = [
        pltpu.VMEM_SHARED((NP, width), jnp.float32),
        pltpu.VMEM((ECH, EB), jnp.int32),
        pltpu.VMEM((ECH, EB), jnp.int32),
        pltpu.VMEM((EB, width), jnp.float32),
        pltpu.VMEM((EB, width), jnp.float32),
        pltpu.SemaphoreType.DMA,
        pltpu.SemaphoreType.DMA,
    ]
    if with_counts:
        out_type.append(jax.ShapeDtypeStruct((NC, N, CW), jnp.float32))
        scratch += [
            pltpu.VMEM_SHARED((NP, CW), jnp.float32),
            pltpu.VMEM((EB, CW), jnp.float32),
        ]

    @functools.partial(
        pl.kernel,
        out_type=out_type,
        mesh=mesh,
        scratch_types=scratch,
        compiler_params=pltpu.CompilerParams(use_tc_tiling_on_sc=tiled),
    )
    def body(table_hbm, src_hbm, dst_hbm, zeros_hbm, zcnt_hbm, ones_hbm, *rest):
        if with_counts:
            (out_hbm, cnt_hbm, acc, src_v, dst_v, rows0, rows1, sem0, sem1,
             cacc, ones_v) = rest
        else:
            out_hbm, acc, src_v, dst_v, rows0, rows1, sem0, sem1 = rest
        c = lax.axis_index("c")
        s = lax.axis_index("s")
        base = jnp.where(c == 0, s * NB0, NS * NB0 + s * NB1)
        my_chunks = jnp.where(c == 0, NB0 // ECH, NB1 // ECH)
        pltpu.sync_copy(zeros_hbm, acc.at[pl.ds(s * RPT, RPT)])
        if with_counts:
            pltpu.sync_copy(zcnt_hbm, cacc.at[pl.ds(s * RPT, RPT)])
            pltpu.sync_copy(ones_hbm, ones_v)
        plsc.subcore_barrier()

        def chunk(ci, carry):
            pltpu.sync_copy(src_hbm.at[pl.ds(base + ci * ECH, ECH)], src_v)
            pltpu.sync_copy(dst_hbm.at[pl.ds(base + ci * ECH, ECH)], dst_v)
            pltpu.async_copy(table_hbm.at[src_v.at[0]], rows0, sem0)
            for jj in range(ECH // 2):
                j0 = 2 * jj
                j1 = j0 + 1
                pltpu.async_copy(table_hbm.at[src_v.at[j1]], rows1, sem1)
                pltpu.make_async_copy(table_hbm.at[src_v.at[j0]], rows0, sem0).wait()
                pltpu.sync_copy(rows0, acc.at[dst_v.at[j0]], add=True)
                if with_counts:
                    pltpu.sync_copy(ones_v, cacc.at[dst_v.at[j0]], add=True)
                if jj < ECH // 2 - 1:
                    pltpu.async_copy(table_hbm.at[src_v.at[j0 + 2]], rows0, sem0)
                pltpu.make_async_copy(table_hbm.at[src_v.at[j1]], rows1, sem1).wait()
                pltpu.sync_copy(rows1, acc.at[dst_v.at[j1]], add=True)
                if with_counts:
                    pltpu.sync_copy(ones_v, cacc.at[dst_v.at[j1]], add=True)
            return carry

        lax.fori_loop(0, my_chunks, chunk, 0)
        plsc.subcore_barrier()

        last = (N // RPT) * RPT
        rem = N - last

        @pl.when(s * RPT < last)
        def _():
            pltpu.sync_copy(acc.at[pl.ds(s * RPT, RPT)],
                            out_hbm.at[c, pl.ds(s * RPT, RPT)])
            if with_counts:
                pltpu.sync_copy(cacc.at[pl.ds(s * RPT, RPT)],
                                cnt_hbm.at[c, pl.ds(s * RPT, RPT)])

        @pl.when(s * RPT >= last)
        def _():
            pltpu.sync_copy(acc.at[pl.ds(last, rem)],
                            out_hbm.at[c, pl.ds(last, rem)])
            if with_counts:
                pltpu.sync_copy(cacc.at[pl.ds(last, rem)],
                                cnt_hbm.at[c, pl.ds(last, rem)])

    return body(table, src2, dst2, zeros, zcnt, ones)


def kernel(x, edge_index, W1_l, W1_r, b1, W2_l, W2_r, b2):
    E = edge_index.shape[1]
    totb = NS * (NB0 + NB1)
    EP = totb * EB
    assert EP >= E
    pad = EP - E
    src2 = jnp.concatenate(
        [edge_index[0], jnp.zeros((pad,), jnp.int32)]).reshape(totb, EB)
    dst2 = jnp.concatenate(
        [edge_index[1], N + jnp.arange(pad, dtype=jnp.int32) % (NP - N)]
    ).reshape(totb, EB)
    zeros1 = jnp.zeros((RPT, W1P), jnp.float32)
    zeros2 = jnp.zeros((RPT, W2P), jnp.float32)
    zcnt = jnp.zeros((RPT, CW), jnp.float32)
    ones = jnp.ones((EB, CW), jnp.float32)

    y1, xr1 = _tc_layer1(x, W1_l, W1_r, b1)
    part1, cntp = _seg_sum(y1, src2, dst2, zeros1, zcnt, ones, W1P, True)
    y2p, hr2, inv = _tc_layer2(part1, cntp, xr1, W2_l, W2_r, b2)
    (part2,) = _seg_sum(y2p, src2, dst2, zeros2, zcnt, ones, W2P, False)
    out = _tc_out(part2, inv, hr2)
    return (out, edge_index)


# speedup vs baseline: 1.0180x; 1.0180x over previous
import functools

import jax
import jax.numpy as jnp
from jax import lax
from jax.experimental import pallas as pl
from jax.experimental.pallas import tpu as pltpu
from jax.experimental.pallas import tpu_sc as plsc

N = 10000
D = 128
H = 128
C = 40

NC = 2
NS = 16
NW = NC * NS
NP = 10240
RPT = NP // NS

W1P = 128
W2P = 48
CW = 8

EB = 128
ECH = 8
NB0 = 144
NB1 = 16

BM = 2000


def _tc_layer1(x, W1_l, W1_r, b1):
    def body(x_ref, wl_ref, wr_ref, b_ref, y1_ref, xr_ref):
        xb = x_ref[...]
        y1_ref[...] = jnp.dot(xb, wl_ref[...], preferred_element_type=jnp.float32)
        xr_ref[...] = jnp.dot(xb, wr_ref[...], preferred_element_type=jnp.float32) + b_ref[...]

    return pl.pallas_call(
        body,
        grid=(N // BM,),
        in_specs=[
            pl.BlockSpec((BM, D), lambda i: (i, 0)),
            pl.BlockSpec((D, H), lambda i: (0, 0)),
            pl.BlockSpec((D, H), lambda i: (0, 0)),
            pl.BlockSpec((1, H), lambda i: (0, 0)),
        ],
        out_specs=[
            pl.BlockSpec((BM, W1P), lambda i: (i, 0)),
            pl.BlockSpec((BM, H), lambda i: (i, 0)),
        ],
        out_shape=[
            jax.ShapeDtypeStruct((N, W1P), jnp.float32),
            jax.ShapeDtypeStruct((N, H), jnp.float32),
        ],
    )(x, W1_l, W1_r, b1.reshape(1, H))


def _tc_layer2(part1, cntp, xr1, W2_l, W2_r, b2):
    def body(p_ref, c_ref, xr_ref, wl_ref, wr_ref, b_ref, y2_ref, hr_ref, inv_ref):
        p = p_ref[0] + p_ref[1]
        cnt = (c_ref[0] + c_ref[1])[:, :1]
        inv = 1.0 / jnp.maximum(cnt, 1.0)
        h = jnp.maximum(p * inv + xr_ref[...], 0.0)
        y2_ref[:, :C] = jnp.dot(h, wl_ref[...], preferred_element_type=jnp.float32)
        y2_ref[:, C:] = jnp.zeros((BM, W2P - C), jnp.float32)
        hr_ref[...] = jnp.dot(h, wr_ref[...], preferred_element_type=jnp.float32) + b_ref[...]
        inv_ref[...] = jnp.broadcast_to(inv, (BM, 8))

    return pl.pallas_call(
        body,
        grid=(N // BM,),
        in_specs=[
            pl.BlockSpec((NC, BM, W1P), lambda i: (0, i, 0)),
            pl.BlockSpec((NC, BM, CW), lambda i: (0, i, 0)),
            pl.BlockSpec((BM, H), lambda i: (i, 0)),
            pl.BlockSpec((H, C), lambda i: (0, 0)),
            pl.BlockSpec((H, C), lambda i: (0, 0)),
            pl.BlockSpec((1, C), lambda i: (0, 0)),
        ],
        out_specs=[
            pl.BlockSpec((BM, W2P), lambda i: (i, 0)),
            pl.BlockSpec((BM, C), lambda i: (i, 0)),
            pl.BlockSpec((BM, 8), lambda i: (i, 0)),
        ],
        out_shape=[
            jax.ShapeDtypeStruct((N, W2P), jnp.float32),
            jax.ShapeDtypeStruct((N, C), jnp.float32),
            jax.ShapeDtypeStruct((N, 8), jnp.float32),
        ],
    )(part1, cntp, xr1, W2_l, W2_r, b2.reshape(1, C))


def _tc_out(part2, inv, hr2):
    def body(p_ref, inv_ref, hr_ref, o_ref):
        p = p_ref[0] + p_ref[1]
        o_ref[...] = p[:, :C] * inv_ref[:, :1] + hr_ref[...]

    return pl.pallas_call(
        body,
        grid=(N // BM,),
        in_specs=[
            pl.BlockSpec((NC, BM, W2P), lambda i: (0, i, 0)),
            pl.BlockSpec((BM, 8), lambda i: (i, 0)),
            pl.BlockSpec((BM, C), lambda i: (i, 0)),
        ],
        out_specs=pl.BlockSpec((BM, C), lambda i: (i, 0)),
        out_shape=jax.ShapeDtypeStruct((N, C), jnp.float32),
    )(part2, inv, hr2)


def _seg_sum(table, src2, dst2, zeros, zcnt, ones, width, with_counts, tiled=False):
    mesh = plsc.VectorSubcoreMesh(
        core_axis_name="c", subcore_axis_name="s", num_cores=NC, num_subcores=NS)

    out_type = [jax.ShapeDtypeStruct((NC, N, width), jnp.float32)]
    scratch = [
        pltpu.VMEM_SHARED((NP, width), jnp.float32),
        pltpu.VMEM((ECH, EB), jnp.int32),
        pltpu.VMEM((ECH, EB), jnp.int32),
        pltpu.VMEM((EB, width), jnp.float32),
        pltpu.VMEM((EB, width), jnp.float32),
        pltpu.SemaphoreType.DMA,
        pltpu.SemaphoreType.DMA,
    ]
    if with_counts:
        out_type.append(jax.ShapeDtypeStruct((NC, N, CW), jnp.float32))
        scratch += [
            pltpu.VMEM_SHARED((NP, CW), jnp.float32),
            pltpu.VMEM((EB, CW), jnp.float32),
        ]

    @functools.partial(
        pl.kernel,
        out_type=out_type,
        mesh=mesh,
        scratch_types=scratch,
        compiler_params=pltpu.CompilerParams(use_tc_tiling_on_sc=tiled),
    )
    def body(table_hbm, src_hbm, dst_hbm, zeros_hbm, zcnt_hbm, ones_hbm, *rest):
        if with_counts:
            (out_hbm, cnt_hbm, acc, src_v, dst_v, rows0, rows1, sem0, sem1,
             cacc, ones_v) = rest
        else:
            out_hbm, acc, src_v, dst_v, rows0, rows1, sem0, sem1 = rest
        c = lax.axis_index("c")
        s = lax.axis_index("s")
        base = jnp.where(c == 0, s * NB0, NS * NB0 + s * NB1)
        my_chunks = jnp.where(c == 0, NB0 // ECH, NB1 // ECH)
        pltpu.sync_copy(zeros_hbm, acc.at[pl.ds(s * RPT, RPT)])
        if with_counts:
            pltpu.sync_copy(zcnt_hbm, cacc.at[pl.ds(s * RPT, RPT)])
            pltpu.sync_copy(ones_hbm, ones_v)
        plsc.subcore_barrier()

        def chunk(ci, carry):
            pltpu.sync_copy(src_hbm.at[pl.ds(base + ci * ECH, ECH)], src_v)
            pltpu.sync_copy(dst_hbm.at[pl.ds(base + ci * ECH, ECH)], dst_v)
            pltpu.async_copy(table_hbm.at[src_v.at[0]], rows0, sem0)
            for jj in range(ECH // 2):
                j0 = 2 * jj
                j1 = j0 + 1
                pltpu.async_copy(table_hbm.at[src_v.at[j1]], rows1, sem1)
                pltpu.make_async_copy(table_hbm.at[src_v.at[j0]], rows0, sem0).wait()
                pltpu.sync_copy(rows0, acc.at[dst_v.at[j0]], add=True)
                if with_counts:
                    pltpu.sync_copy(ones_v, cacc.at[dst_v.at[j0]], add=True)
                if jj < ECH // 2 - 1:
                    pltpu.async_copy(table_hbm.at[src_v.at[j0 + 2]], rows0, sem0)
                pltpu.make_async_copy(table_hbm.at[src_v.at[j1]], rows1, sem1).wait()
                pltpu.sync_copy(rows1, acc.at[dst_v.at[j1]], add=True)
                if with_counts:
                    pltpu.sync_copy(ones_v, cacc.at[dst_v.at[j1]], add=True)
            return carry

        lax.fori_loop(0, my_chunks, chunk, 0)
        plsc.subcore_barrier()

        last = (N // RPT) * RPT
        rem = N - last

        @pl.when(s * RPT < last)
        def _():
            pltpu.sync_copy(acc.at[pl.ds(s * RPT, RPT)],
                            out_hbm.at[c, pl.ds(s * RPT, RPT)])
            if with_counts:
                pltpu.sync_copy(cacc.at[pl.ds(s * RPT, RPT)],
                                cnt_hbm.at[c, pl.ds(s * RPT, RPT)])

        @pl.when(s * RPT >= last)
        def _():
            pltpu.sync_copy(acc.at[pl.ds(last, rem)],
                            out_hbm.at[c, pl.ds(last, rem)])
            if with_counts:
                pltpu.sync_copy(cacc.at[pl.ds(last, rem)],
                                cnt_hbm.at[c, pl.ds(last, rem)])

    return body(table, src2, dst2, zeros, zcnt, ones)


def kernel(x, edge_index, W1_l, W1_r, b1, W2_l, W2_r, b2):
    E = edge_index.shape[1]
    totb = NS * (NB0 + NB1)
    EP = totb * EB
    assert EP >= E
    pad = EP - E
    src2 = jnp.concatenate(
        [edge_index[0], jnp.zeros((pad,), jnp.int32)]).reshape(totb, EB)
    dst2 = jnp.concatenate(
        [edge_index[1], N + jnp.arange(pad, dtype=jnp.int32) % (NP - N)]
    ).reshape(totb, EB)
    zeros1 = jnp.zeros((RPT, W1P), jnp.float32)
    zeros2 = jnp.zeros((RPT, W2P), jnp.float32)
    zcnt = jnp.zeros((RPT, CW), jnp.float32)
    ones = jnp.ones((EB, CW), jnp.float32)

    y1, xr1 = _tc_layer1(x, W1_l, W1_r, b1)
    part1, cntp = _seg_sum(y1, src2, dst2, zeros1, zcnt, ones, W1P, True)
    y2p, hr2, inv = _tc_layer2(part1, cntp, xr1, W2_l, W2_r, b2)
    (part2,) = _seg_sum(y2p, src2, dst2, zeros2, zcnt, ones, W2P, False)
    out = _tc_out(part2, inv, hr2)
    return (out, edge_index)
